# baseline (speedup 1.0000x reference)
import functools

import jax
import jax.numpy as jnp
from jax import lax
from jax.experimental import pallas as pl
from jax.experimental.pallas import tpu as pltpu
from jax.experimental.pallas import tpu_sc as plsc

_N = 10000
_F = 128
_E = 320000
_CHUNK = 128
_TILES = 16
_CPT = 160
_NCHUNK = _CPT * _TILES
_EPAD = _NCHUNK * _CHUNK
_ACC_ROWS = _N + 16
_ROWS_MAIN = 624
_ZROWS = 48
_GRP = 8
_NGRP = _CPT // _GRP


def _seg_body(x_hbm, dst_hbm, src_hbm, out_hbm, dstb0, dstb1, srcb0, srcb1,
              dst_v, rows, zbuf, acc, gsem0, gsem1):
    c = lax.axis_index("c")
    s = lax.axis_index("s")

    zv = jnp.zeros((16,), jnp.float32)

    def zstore(i, carry):
        zbuf[i // 8, pl.ds((i % 8) * 16, 16)] = zv
        return carry

    lax.fori_loop(0, _ZROWS * 8, zstore, 0)

    row0 = s * _ROWS_MAIN + 8 * jnp.minimum(s, 2)

    def zcopy(k, carry):
        pltpu.sync_copy(zbuf, acc.at[pl.ds(row0 + k * _ZROWS, _ZROWS)])
        return carry

    lax.fori_loop(0, _ROWS_MAIN // _ZROWS, zcopy, 0)

    @pl.when(s < 2)
    def _():
        pltpu.sync_copy(zbuf.at[pl.ds(0, 8)], acc.at[pl.ds(row0 + _ROWS_MAIN, 8)])

    edge0 = c * _EPAD + s * _CPT * _CHUNK
    _GW = _GRP * _CHUNK

    def load_grp(g, db, sb):
        pltpu.sync_copy(dst_hbm.at[pl.ds(edge0 + g * _GW, _GW)], db)
        pltpu.sync_copy(src_hbm.at[pl.ds(edge0 + g * _GW, _GW)], sb)

    load_grp(0, dstb0, srcb0)
    plsc.subcore_barrier()

    sems = (gsem0, gsem1)
    pltpu.async_copy(x_hbm.at[srcb0.at[pl.ds(0, _CHUNK)]], rows.at[0], gsem0)
    pltpu.async_copy(x_hbm.at[srcb0.at[pl.ds(_CHUNK, _CHUNK)]], rows.at[1], gsem1)

    def do_chunk(i, j, dbuf, j8, sbuf_pf, j8_pf):
        for t in range(_CHUNK // 16):
            dst_v[pl.ds(t * 16, 16)] = dbuf[pl.ds(j8 * _CHUNK + t * 16, 16)]
        pltpu.make_async_copy(
            x_hbm.at[sbuf_pf.at[pl.ds(j8_pf * _CHUNK, _CHUNK)]], rows.at[j],
            sems[j]).wait()
        pltpu.sync_copy(rows.at[j], acc.at[dst_v], add=True)

        @pl.when(i + 2 < _CPT)
        def _():
            pltpu.async_copy(
                x_hbm.at[sbuf_pf.at[pl.ds(j8_pf * _CHUNK, _CHUNK)]],
                rows.at[j], sems[j])

    def pair_body(gg, carry):
        g0 = gg * 2
        load_grp(g0 + 1, dstb1, srcb1)
        for j8 in range(_GRP):
            i = g0 * _GRP + j8
            pf_buf = srcb0 if j8 < _GRP - 2 else srcb1
            do_chunk(i, j8 % 2, dstb0, j8, pf_buf, (j8 + 2) % _GRP)
        @pl.when(g0 + 2 < _NGRP)
        def _():
            load_grp(g0 + 2, dstb0, srcb0)
        for j8 in range(_GRP):
            i = (g0 + 1) * _GRP + j8
            pf_buf = srcb1 if j8 < _GRP - 2 else srcb0
            do_chunk(i, j8 % 2, dstb1, j8, pf_buf, (j8 + 2) % _GRP)
        return carry

    lax.fori_loop(0, _NGRP // 2, pair_body, 0)
    plsc.subcore_barrier()

    pltpu.sync_copy(acc.at[pl.ds(row0, _ROWS_MAIN)],
                    out_hbm.at[c, pl.ds(row0, _ROWS_MAIN)])

    @pl.when(s < 2)
    def _():
        pltpu.sync_copy(acc.at[pl.ds(row0 + _ROWS_MAIN, 8)],
                        out_hbm.at[c, pl.ds(row0 + _ROWS_MAIN, 8)])


def _make_seg():
    mesh = plsc.VectorSubcoreMesh(core_axis_name="c", subcore_axis_name="s")
    return pl.kernel(
        _seg_body,
        out_type=jax.ShapeDtypeStruct((2, _N, _F), jnp.float32),
        mesh=mesh,
        scratch_types=[
            pltpu.VMEM((_GRP * _CHUNK,), jnp.int32),
            pltpu.VMEM((_GRP * _CHUNK,), jnp.int32),
            pltpu.VMEM((_GRP * _CHUNK,), jnp.int32),
            pltpu.VMEM((_GRP * _CHUNK,), jnp.int32),
            pltpu.VMEM((_CHUNK,), jnp.int32),
            pltpu.VMEM((2, _CHUNK, _F), jnp.float32),
            pltpu.VMEM((_ZROWS, _F), jnp.float32),
            pltpu.VMEM_SHARED((_ACC_ROWS, _F), jnp.float32),
            pltpu.SemaphoreType.DMA,
            pltpu.SemaphoreType.DMA,
        ],
        name="segment_sums_sc",
    )


def _dense_body(x_ref, nsum_ref, psum_ref, wn_ref, wp_ref, b_ref, g_ref,
                beta_ref, gk_ref, grk_ref, gb_ref, o_ref):
    x = x_ref[...]
    aggre = jnp.dot(nsum_ref[...], wn_ref[...], preferred_element_type=jnp.float32)
    aggre = aggre + jnp.dot(psum_ref[...], wp_ref[...], preferred_element_type=jnp.float32)
    aggre = aggre + b_ref[...] + x
    a = jnp.maximum(aggre, 0.0)
    mean = jnp.mean(a, axis=0, keepdims=True)
    var = jnp.mean((a - mean) * (a - mean), axis=0, keepdims=True)
    a = (a - mean) / jnp.sqrt(var + 1e-3) * g_ref[...] + beta_ref[...]
    mx = jnp.dot(a, gk_ref[...], preferred_element_type=jnp.float32) + gb_ref[0:1, :]
    mi = jnp.dot(x, grk_ref[...], preferred_element_type=jnp.float32) + gb_ref[1:2, :]
    z = jax.nn.sigmoid(mx[:, :_F] + mi[:, :_F])
    r = jax.nn.sigmoid(mx[:, _F:2 * _F] + mi[:, _F:2 * _F])
    h = jnp.tanh(mx[:, 2 * _F:] + r * mi[:, 2 * _F:])
    o_ref[...] = z * x + (1.0 - z) * h


def _make_dense(interpret=False):
    return pl.pallas_call(
        _dense_body,
        out_shape=jax.ShapeDtypeStruct((_N, _F), jnp.float32),
        interpret=interpret,
        name="dense_tail_tc",
    )


@functools.cache
def _get_seg():
    return _make_seg()


@functools.cache
def _get_dense():
    return _make_dense()


def kernel(x, pairs_prev, pairs_next, w_next, w_prev, b, bn_gamma, bn_beta,
           gru_kernel, gru_rec_kernel, gru_bias):
    npad = _EPAD - _E
    dpad = jnp.full((npad,), _N, jnp.int32)
    spad = jnp.zeros((npad,), jnp.int32)
    dst = jnp.concatenate([pairs_prev[:, 0], dpad, pairs_next[:, 0], dpad])
    src = jnp.concatenate([pairs_prev[:, 1], spad, pairs_next[:, 1], spad])
    sums = _get_seg()(x, dst, src)
    prev_sumx = sums[0]
    next_sumx = sums[1]
    return _get_dense()(x, next_sumx, prev_sumx, w_next, w_prev, b,
                        bn_gamma.reshape(1, _F), bn_beta.reshape(1, _F),
                        gru_kernel, gru_rec_kernel, gru_bias)
